# Initial kernel scaffold; baseline (speedup 1.0000x reference)
import jax
import jax.numpy as jnp
from jax.experimental import pallas as pl


def kernel(x, W_gate, expert_gate_up, expert_down):
    raise NotImplementedError("write your pallas kernel here")


# speedup vs baseline: 4.3829x; 4.3829x over previous
import functools

import jax
import jax.numpy as jnp
from jax.experimental import pallas as pl
from jax.experimental.pallas import tpu as pltpu

T = 2048
D = 1024
I = 2048
E = 8
K = 2
NP = T * K
BLK = 256
NBLK = NP // BLK + E - 1
NPAD = NBLK * BLK
IC = 512
NIC = I // IC
CH = 256
TB = 256


def _router_kernel(x_ref, wg_ref, ti_ref, tw_ref, cnt_ref, pos_ref, aux_ref,
                   ohs_ref):
    x = x_ref[...]
    wg = wg_ref[...]
    logits = jax.lax.dot_general(x, wg, (((1,), (1,)), ((), ())),
                                 preferred_element_type=jnp.float32)
    m = jnp.max(logits, axis=1, keepdims=True)
    ex = jnp.exp(logits - m)
    probs = ex / jnp.sum(ex, axis=1, keepdims=True)

    eids = jax.lax.broadcasted_iota(jnp.int32, (T, E), 1)
    m1 = jnp.max(probs, axis=1, keepdims=True)
    i1 = jnp.min(jnp.where(probs >= m1, eids, E), axis=1, keepdims=True)
    oh1 = (eids == i1)
    probs2 = jnp.where(oh1, -jnp.inf, probs)
    m2 = jnp.max(probs2, axis=1, keepdims=True)
    i2 = jnp.min(jnp.where(probs2 >= m2, eids, E), axis=1, keepdims=True)
    oh2 = (eids == i2)
    denom = m1 + m2 + 1e-9
    w1 = m1 / denom
    w2 = m2 / denom

    oh1f = oh1.astype(jnp.float32)
    oh2f = oh2.astype(jnp.float32)
    ohs = oh1f + oh2f
    cnt = jnp.sum(ohs, axis=0, keepdims=True)
    p_mean = jnp.mean(probs, axis=0, keepdims=True)
    aux_ref[...] = E * jnp.sum((cnt / T) * p_mean, axis=1, keepdims=True)
    cnt_i = cnt.astype(jnp.int32)
    cnt_ref[...] = cnt_i

    pcnt = ((cnt + (BLK - 1)) // BLK).astype(jnp.float32) * BLK
    r8 = jax.lax.broadcasted_iota(jnp.int32, (E, E), 0)
    c8 = jax.lax.broadcasted_iota(jnp.int32, (E, E), 1)
    lt8 = (r8 < c8).astype(jnp.float32)
    pad_off = jax.lax.dot_general(pcnt, lt8, (((1,), (0,)), ((), ())),
                                  preferred_element_type=jnp.float32)

    ohs_ref[...] = ohs
    rr = jax.lax.broadcasted_iota(jnp.int32, (CH, CH), 0)
    cc = jax.lax.broadcasted_iota(jnp.int32, (CH, CH), 1)
    ltri = (cc < rr).astype(jnp.float32)

    def scan_body(c, carry):
        blk = ohs_ref[pl.ds(c * CH, CH), :]
        pre = jax.lax.dot_general(ltri, blk, (((1,), (0,)), ((), ())),
                                  preferred_element_type=jnp.float32)
        ohs_ref[pl.ds(c * CH, CH), :] = pre + carry
        return carry + jnp.sum(blk, axis=0, keepdims=True)

    jax.lax.fori_loop(0, T // CH, scan_body, jnp.zeros((1, E), jnp.float32))
    cex = ohs_ref[...]

    sel_off1 = jnp.sum(oh1f * pad_off, axis=1, keepdims=True)
    sel_off2 = jnp.sum(oh2f * pad_off, axis=1, keepdims=True)
    rank1 = jnp.sum(oh1f * cex, axis=1, keepdims=True)
    rank2 = jnp.sum(oh2f * cex, axis=1, keepdims=True)
    pos1 = (sel_off1 + rank1).astype(jnp.int32)
    pos2 = (sel_off2 + rank2).astype(jnp.int32)

    ti_ref[...] = jnp.concatenate([i1, i2], axis=1)
    tw_ref[...] = jnp.concatenate([w1, w2], axis=1)
    pos_ref[...] = jnp.concatenate([pos1, pos2], axis=1)


def _router_call(x_flat, w_gate, interpret=False):
    return pl.pallas_call(
        _router_kernel,
        out_shape=[
            jax.ShapeDtypeStruct((T, K), jnp.int32),
            jax.ShapeDtypeStruct((T, K), jnp.float32),
            jax.ShapeDtypeStruct((1, E), jnp.int32),
            jax.ShapeDtypeStruct((T, K), jnp.int32),
            jax.ShapeDtypeStruct((1, 1), jnp.float32),
        ],
        scratch_shapes=[pltpu.VMEM((T, E), jnp.float32)],
        interpret=interpret,
    )(x_flat, w_gate)


def _ffn_kernel(be_ref, nb_ref, tok_ref, x_ref, gate_ref, up_ref, down_ref,
                y_ref, xg_ref):
    b = pl.program_id(0)
    ic = pl.program_id(1)

    @pl.when(b < nb_ref[0])
    def _active():
        @pl.when(ic == 0)
        def _gather():
            def body(r, _):
                t = tok_ref[b * BLK + r]
                xg_ref[pl.ds(r, 1), :] = x_ref[pl.ds(t, 1), :]
                return 0
            jax.lax.fori_loop(0, BLK, body, 0)

        xg = xg_ref[...]
        g = jax.lax.dot_general(xg, gate_ref[0], (((1,), (1,)), ((), ())),
                                preferred_element_type=jnp.float32)
        u = jax.lax.dot_general(xg, up_ref[0], (((1,), (1,)), ((), ())),
                                preferred_element_type=jnp.float32)
        act = (g / (1.0 + jnp.exp(-g))) * u
        part = jax.lax.dot_general(act, down_ref[0], (((1,), (1,)), ((), ())),
                                   preferred_element_type=jnp.float32)

        @pl.when(ic == 0)
        def _init():
            y_ref[...] = part

        @pl.when(ic > 0)
        def _acc():
            y_ref[...] += part


def _ffn_call(be, nb, tok_pad, x_flat, egu, ed, interpret=False):
    spec = pltpu.PrefetchScalarGridSpec(
        num_scalar_prefetch=3,
        grid=(NBLK, NIC),
        in_specs=[
            pl.BlockSpec((T, D), lambda b, ic, be, nb, tok: (0, 0)),
            pl.BlockSpec((1, IC, D), lambda b, ic, be, nb, tok: (be[b], ic, 0)),
            pl.BlockSpec((1, IC, D),
                         lambda b, ic, be, nb, tok: (be[b], NIC + ic, 0)),
            pl.BlockSpec((1, D, IC), lambda b, ic, be, nb, tok: (be[b], 0, ic)),
        ],
        out_specs=pl.BlockSpec((BLK, D), lambda b, ic, be, nb, tok: (b, 0)),
        scratch_shapes=[pltpu.VMEM((BLK, D), jnp.float32)],
    )
    return pl.pallas_call(
        _ffn_kernel,
        grid_spec=spec,
        out_shape=jax.ShapeDtypeStruct((NPAD, D), jnp.float32),
        interpret=interpret,
    )(be, nb, tok_pad, x_flat, egu, egu, ed)


def _combine_kernel(pos_ref, twb_ref, y_ref, out_ref):
    tb = pl.program_id(0)

    def body(r, _):
        g = tb * TB + r
        p0 = pos_ref[2 * g]
        p1 = pos_ref[2 * g + 1]
        w0 = jax.lax.bitcast_convert_type(twb_ref[2 * g], jnp.float32)
        w1 = jax.lax.bitcast_convert_type(twb_ref[2 * g + 1], jnp.float32)
        out_ref[pl.ds(r, 1), :] = (y_ref[pl.ds(p0, 1), :] * w0
                                   + y_ref[pl.ds(p1, 1), :] * w1)
        return 0

    jax.lax.fori_loop(0, TB, body, 0)


def _combine_call(pos_flat, tw_flat_bits, y_pad, interpret=False):
    spec = pltpu.PrefetchScalarGridSpec(
        num_scalar_prefetch=2,
        grid=(T // TB,),
        in_specs=[pl.BlockSpec((NPAD, D), lambda tb, pos, tw: (0, 0))],
        out_specs=pl.BlockSpec((TB, D), lambda tb, pos, tw: (tb, 0)),
    )
    return pl.pallas_call(
        _combine_kernel,
        grid_spec=spec,
        out_shape=jax.ShapeDtypeStruct((T, D), jnp.float32),
        interpret=interpret,
    )(pos_flat, tw_flat_bits, y_pad)


def _moe(x, w_gate, egu, ed, interpret=False):
    b_, s_, d_ = x.shape
    x_flat = x.reshape(T, D)
    ti, tw, cnt, pos, aux = _router_call(x_flat, w_gate, interpret=interpret)

    pb = (cnt.reshape(E) + (BLK - 1)) // BLK
    ends = jnp.cumsum(pb)
    nblk_act = ends[E - 1].astype(jnp.int32).reshape(1)
    bids = jnp.arange(NBLK, dtype=jnp.int32)
    be = jnp.minimum(jnp.sum((bids[:, None] >= ends[None, :]).astype(jnp.int32),
                             axis=1), E - 1).astype(jnp.int32)
    pos_flat = pos.reshape(NP)
    tok_src = (jnp.arange(NP, dtype=jnp.int32) // K)
    tok_pad = jnp.zeros((NPAD,), jnp.int32).at[pos_flat].set(tok_src)

    y_pad = _ffn_call(be, nblk_act, tok_pad, x_flat, egu, ed,
                      interpret=interpret)

    tw_bits = jax.lax.bitcast_convert_type(tw.reshape(NP), jnp.int32)
    out = _combine_call(pos_flat, tw_bits, y_pad, interpret=interpret)
    return out.reshape(b_, s_, d_), aux[0, 0]


def kernel(x, W_gate, expert_gate_up, expert_down):
    return _moe(x, W_gate, expert_gate_up, expert_down)


# speedup vs baseline: 4.9979x; 1.1403x over previous
import functools

import jax
import jax.numpy as jnp
from jax.experimental import pallas as pl
from jax.experimental.pallas import tpu as pltpu

T = 2048
D = 1024
I = 2048
E = 8
K = 2
NP = T * K
BLK = 256
NBLK = NP // BLK + E - 1
NPAD = NBLK * BLK
IC = 512
NIC = I // IC
CH = 256
TB = 256


def _router_kernel(x_ref, wg_ref, ti_ref, tw_ref, cnt_ref, pos_ref, aux_ref,
                   ohs_ref):
    x = x_ref[...]
    wg = wg_ref[...]
    logits = jax.lax.dot_general(x, wg, (((1,), (1,)), ((), ())),
                                 preferred_element_type=jnp.float32)
    m = jnp.max(logits, axis=1, keepdims=True)
    ex = jnp.exp(logits - m)
    probs = ex / jnp.sum(ex, axis=1, keepdims=True)

    eids = jax.lax.broadcasted_iota(jnp.int32, (T, E), 1)
    m1 = jnp.max(probs, axis=1, keepdims=True)
    i1 = jnp.min(jnp.where(probs >= m1, eids, E), axis=1, keepdims=True)
    oh1 = (eids == i1)
    probs2 = jnp.where(oh1, -jnp.inf, probs)
    m2 = jnp.max(probs2, axis=1, keepdims=True)
    i2 = jnp.min(jnp.where(probs2 >= m2, eids, E), axis=1, keepdims=True)
    oh2 = (eids == i2)
    denom = m1 + m2 + 1e-9
    w1 = m1 / denom
    w2 = m2 / denom

    oh1f = oh1.astype(jnp.float32)
    oh2f = oh2.astype(jnp.float32)
    ohs = oh1f + oh2f
    cnt = jnp.sum(ohs, axis=0, keepdims=True)
    p_mean = jnp.mean(probs, axis=0, keepdims=True)
    aux_ref[...] = E * jnp.sum((cnt / T) * p_mean, axis=1, keepdims=True)
    cnt_i = cnt.astype(jnp.int32)
    cnt_ref[...] = cnt_i

    pcnt = ((cnt + (BLK - 1)) // BLK).astype(jnp.float32) * BLK
    r8 = jax.lax.broadcasted_iota(jnp.int32, (E, E), 0)
    c8 = jax.lax.broadcasted_iota(jnp.int32, (E, E), 1)
    lt8 = (r8 < c8).astype(jnp.float32)
    pad_off = jax.lax.dot_general(pcnt, lt8, (((1,), (0,)), ((), ())),
                                  preferred_element_type=jnp.float32)

    ohs_ref[...] = ohs
    rr = jax.lax.broadcasted_iota(jnp.int32, (CH, CH), 0)
    cc = jax.lax.broadcasted_iota(jnp.int32, (CH, CH), 1)
    ltri = (cc < rr).astype(jnp.float32)

    def scan_body(c, carry):
        blk = ohs_ref[pl.ds(c * CH, CH), :]
        pre = jax.lax.dot_general(ltri, blk, (((1,), (0,)), ((), ())),
                                  preferred_element_type=jnp.float32)
        ohs_ref[pl.ds(c * CH, CH), :] = pre + carry
        return carry + jnp.sum(blk, axis=0, keepdims=True)

    jax.lax.fori_loop(0, T // CH, scan_body, jnp.zeros((1, E), jnp.float32))
    cex = ohs_ref[...]

    sel_off1 = jnp.sum(oh1f * pad_off, axis=1, keepdims=True)
    sel_off2 = jnp.sum(oh2f * pad_off, axis=1, keepdims=True)
    rank1 = jnp.sum(oh1f * cex, axis=1, keepdims=True)
    rank2 = jnp.sum(oh2f * cex, axis=1, keepdims=True)
    pos1 = (sel_off1 + rank1).astype(jnp.int32)
    pos2 = (sel_off2 + rank2).astype(jnp.int32)

    ti_ref[...] = jnp.concatenate([i1, i2], axis=1)
    tw_ref[...] = jnp.concatenate([w1, w2], axis=1)
    pos_ref[...] = jnp.concatenate([pos1, pos2], axis=1)


def _router_call(x_flat, w_gate, interpret=False):
    return pl.pallas_call(
        _router_kernel,
        out_shape=[
            jax.ShapeDtypeStruct((T, K), jnp.int32),
            jax.ShapeDtypeStruct((T, K), jnp.float32),
            jax.ShapeDtypeStruct((1, E), jnp.int32),
            jax.ShapeDtypeStruct((T, K), jnp.int32),
            jax.ShapeDtypeStruct((1, 1), jnp.float32),
        ],
        scratch_shapes=[pltpu.VMEM((T, E), jnp.float32)],
        interpret=interpret,
    )(x_flat, w_gate)


def _ffn_kernel(be_ref, nb_ref, tok_ref, x_ref, gu_ref, down_ref,
                y_ref, xg_ref):
    b = pl.program_id(0)

    @pl.when(b < nb_ref[0])
    def _active():
        def body(r, _):
            t = tok_ref[b * BLK + r]
            xg_ref[pl.ds(r, 1), :] = x_ref[pl.ds(t, 1), :]
            return 0
        jax.lax.fori_loop(0, BLK, body, 0)

        xg = xg_ref[...].astype(jnp.bfloat16)
        h = jax.lax.dot_general(xg, gu_ref[0], (((1,), (1,)), ((), ())),
                                preferred_element_type=jnp.float32)
        g = h[:, :I]
        u = h[:, I:]
        act = ((g / (1.0 + jnp.exp(-g))) * u).astype(jnp.bfloat16)
        y_ref[...] = jax.lax.dot_general(
            act, down_ref[0], (((1,), (1,)), ((), ())),
            preferred_element_type=jnp.float32)


def _ffn_call(be, nb, tok_pad, x_bf, egu, ed, interpret=False):
    spec = pltpu.PrefetchScalarGridSpec(
        num_scalar_prefetch=3,
        grid=(NBLK,),
        in_specs=[
            pl.BlockSpec((T, D), lambda b, be, nb, tok: (0, 0)),
            pl.BlockSpec((1, 2 * I, D), lambda b, be, nb, tok: (be[b], 0, 0)),
            pl.BlockSpec((1, D, I), lambda b, be, nb, tok: (be[b], 0, 0)),
        ],
        out_specs=pl.BlockSpec((BLK, D), lambda b, be, nb, tok: (b, 0)),
        scratch_shapes=[pltpu.VMEM((BLK, D), jnp.float32)],
    )
    return pl.pallas_call(
        _ffn_kernel,
        grid_spec=spec,
        out_shape=jax.ShapeDtypeStruct((NPAD, D), jnp.float32),
        interpret=interpret,
    )(be, nb, tok_pad, x_bf, egu, ed)


def _combine_kernel(pos_ref, twb_ref, y_ref, out_ref):
    tb = pl.program_id(0)

    def body(r, _):
        g = tb * TB + r
        p0 = pos_ref[2 * g]
        p1 = pos_ref[2 * g + 1]
        w0 = jax.lax.bitcast_convert_type(twb_ref[2 * g], jnp.float32)
        w1 = jax.lax.bitcast_convert_type(twb_ref[2 * g + 1], jnp.float32)
        out_ref[pl.ds(r, 1), :] = (y_ref[pl.ds(p0, 1), :] * w0
                                   + y_ref[pl.ds(p1, 1), :] * w1)
        return 0

    jax.lax.fori_loop(0, TB, body, 0)


def _combine_call(pos_flat, tw_flat_bits, y_pad, interpret=False):
    spec = pltpu.PrefetchScalarGridSpec(
        num_scalar_prefetch=2,
        grid=(T // TB,),
        in_specs=[pl.BlockSpec((NPAD, D), lambda tb, pos, tw: (0, 0))],
        out_specs=pl.BlockSpec((TB, D), lambda tb, pos, tw: (tb, 0)),
    )
    return pl.pallas_call(
        _combine_kernel,
        grid_spec=spec,
        out_shape=jax.ShapeDtypeStruct((T, D), jnp.float32),
        interpret=interpret,
    )(pos_flat, tw_flat_bits, y_pad)


def _moe(x, w_gate, egu, ed, interpret=False):
    b_, s_, d_ = x.shape
    x_flat = x.reshape(T, D)
    ti, tw, cnt, pos, aux = _router_call(x_flat, w_gate, interpret=interpret)

    pb = (cnt.reshape(E) + (BLK - 1)) // BLK
    ends = jnp.cumsum(pb)
    nblk_act = ends[E - 1].astype(jnp.int32).reshape(1)
    bids = jnp.arange(NBLK, dtype=jnp.int32)
    eb = jnp.minimum(jnp.sum((bids[:, None] >= ends[None, :]).astype(jnp.int32),
                             axis=1), E - 1).astype(jnp.int32)
    last_e = jnp.max(jnp.where(bids < nblk_act[0], eb, -1))
    be = jnp.where(bids < nblk_act[0], eb, last_e).astype(jnp.int32)
    pos_flat = pos.reshape(NP)
    tok_src = (jnp.arange(NP, dtype=jnp.int32) // K)
    tok_pad = jnp.zeros((NPAD,), jnp.int32).at[pos_flat].set(tok_src)

    y_pad = _ffn_call(be, nblk_act, tok_pad, x_flat,
                      egu.astype(jnp.bfloat16), ed.astype(jnp.bfloat16),
                      interpret=interpret)

    tw_bits = jax.lax.bitcast_convert_type(tw.reshape(NP), jnp.int32)
    out = _combine_call(pos_flat, tw_bits, y_pad, interpret=interpret)
    return out.reshape(b_, s_, d_), aux[0, 0]


def kernel(x, W_gate, expert_gate_up, expert_down):
    return _moe(x, W_gate, expert_gate_up, expert_down)


# speedup vs baseline: 5.5271x; 1.1059x over previous
import functools

import jax
import jax.numpy as jnp
from jax.experimental import pallas as pl
from jax.experimental.pallas import tpu as pltpu

T = 2048
D = 1024
I = 2048
E = 8
K = 2
NP = T * K
BLK = 256
NBLK = NP // BLK + E - 1
NPAD = NBLK * BLK
IC = 512
NIC = I // IC
CH = 256
TB = 256


def _router_kernel(x_ref, wg_ref, ti_ref, tw_ref, cnt_ref, pos_ref, aux_ref,
                   ohs_ref):
    x = x_ref[...]
    wg = wg_ref[...]
    logits = jax.lax.dot_general(x, wg, (((1,), (1,)), ((), ())),
                                 preferred_element_type=jnp.float32)
    m = jnp.max(logits, axis=1, keepdims=True)
    ex = jnp.exp(logits - m)
    probs = ex / jnp.sum(ex, axis=1, keepdims=True)

    eids = jax.lax.broadcasted_iota(jnp.int32, (T, E), 1)
    m1 = jnp.max(probs, axis=1, keepdims=True)
    i1 = jnp.min(jnp.where(probs >= m1, eids, E), axis=1, keepdims=True)
    oh1 = (eids == i1)
    probs2 = jnp.where(oh1, -jnp.inf, probs)
    m2 = jnp.max(probs2, axis=1, keepdims=True)
    i2 = jnp.min(jnp.where(probs2 >= m2, eids, E), axis=1, keepdims=True)
    oh2 = (eids == i2)
    denom = m1 + m2 + 1e-9
    w1 = m1 / denom
    w2 = m2 / denom

    oh1f = oh1.astype(jnp.float32)
    oh2f = oh2.astype(jnp.float32)
    ohs = oh1f + oh2f
    cnt = jnp.sum(ohs, axis=0, keepdims=True)
    p_mean = jnp.mean(probs, axis=0, keepdims=True)
    aux_ref[...] = E * jnp.sum((cnt / T) * p_mean, axis=1, keepdims=True)
    cnt_i = cnt.astype(jnp.int32)
    cnt_ref[...] = cnt_i

    pcnt = ((cnt + (BLK - 1)) // BLK).astype(jnp.float32) * BLK
    r8 = jax.lax.broadcasted_iota(jnp.int32, (E, E), 0)
    c8 = jax.lax.broadcasted_iota(jnp.int32, (E, E), 1)
    lt8 = (r8 < c8).astype(jnp.float32)
    pad_off = jax.lax.dot_general(pcnt, lt8, (((1,), (0,)), ((), ())),
                                  preferred_element_type=jnp.float32)

    ohs_ref[...] = ohs
    rr = jax.lax.broadcasted_iota(jnp.int32, (CH, CH), 0)
    cc = jax.lax.broadcasted_iota(jnp.int32, (CH, CH), 1)
    ltri = (cc < rr).astype(jnp.float32)

    def scan_body(c, carry):
        blk = ohs_ref[pl.ds(c * CH, CH), :]
        pre = jax.lax.dot_general(ltri, blk, (((1,), (0,)), ((), ())),
                                  preferred_element_type=jnp.float32)
        ohs_ref[pl.ds(c * CH, CH), :] = pre + carry
        return carry + jnp.sum(blk, axis=0, keepdims=True)

    jax.lax.fori_loop(0, T // CH, scan_body, jnp.zeros((1, E), jnp.float32))
    cex = ohs_ref[...]

    sel_off1 = jnp.sum(oh1f * pad_off, axis=1, keepdims=True)
    sel_off2 = jnp.sum(oh2f * pad_off, axis=1, keepdims=True)
    rank1 = jnp.sum(oh1f * cex, axis=1, keepdims=True)
    rank2 = jnp.sum(oh2f * cex, axis=1, keepdims=True)
    pos1 = (sel_off1 + rank1).astype(jnp.int32)
    pos2 = (sel_off2 + rank2).astype(jnp.int32)

    ti_ref[...] = jnp.concatenate([i1, i2], axis=1)
    tw_ref[...] = jnp.concatenate([w1, w2], axis=1)
    pos_ref[...] = jnp.concatenate([pos1, pos2], axis=1)


def _router_call(x_flat, w_gate, interpret=False):
    return pl.pallas_call(
        _router_kernel,
        out_shape=[
            jax.ShapeDtypeStruct((T, K), jnp.int32),
            jax.ShapeDtypeStruct((T, K), jnp.float32),
            jax.ShapeDtypeStruct((1, E), jnp.int32),
            jax.ShapeDtypeStruct((T, K), jnp.int32),
            jax.ShapeDtypeStruct((1, 1), jnp.float32),
        ],
        scratch_shapes=[pltpu.VMEM((T, E), jnp.float32)],
        interpret=interpret,
    )(x_flat, w_gate)


def _ffn_kernel(be_ref, nb_ref, tok_ref, x_ref, gu_ref, down_ref,
                y_ref, xg_ref):
    b = pl.program_id(0)

    @pl.when(b < nb_ref[0])
    def _active():
        def body(r, _):
            t = tok_ref[b * BLK + r]
            xg_ref[pl.ds(r, 1), :] = x_ref[pl.ds(t, 1), :]
            return 0
        jax.lax.fori_loop(0, BLK, body, 0)

        xg = xg_ref[...]
        h = jax.lax.dot_general(xg, gu_ref[0], (((1,), (1,)), ((), ())),
                                preferred_element_type=jnp.float32)
        g = h[:, :I]
        u = h[:, I:]
        act = (g / (1.0 + jnp.exp(-g))) * u
        y_ref[...] = jax.lax.dot_general(
            act, down_ref[0], (((1,), (1,)), ((), ())),
            preferred_element_type=jnp.float32)


def _ffn_call(be, nb, tok_pad, x_bf, egu, ed, interpret=False):
    spec = pltpu.PrefetchScalarGridSpec(
        num_scalar_prefetch=3,
        grid=(NBLK,),
        in_specs=[
            pl.BlockSpec((T, D), lambda b, be, nb, tok: (0, 0)),
            pl.BlockSpec((1, 2 * I, D), lambda b, be, nb, tok: (be[b], 0, 0),
                         pipeline_mode=pl.Buffered(buffer_count=1)),
            pl.BlockSpec((1, D, I), lambda b, be, nb, tok: (be[b], 0, 0),
                         pipeline_mode=pl.Buffered(buffer_count=1)),
        ],
        out_specs=pl.BlockSpec((BLK, D), lambda b, be, nb, tok: (b, 0)),
        scratch_shapes=[pltpu.VMEM((BLK, D), jnp.float32)],
    )
    return pl.pallas_call(
        _ffn_kernel,
        grid_spec=spec,
        out_shape=jax.ShapeDtypeStruct((NPAD, D), jnp.float32),
        interpret=interpret,
    )(be, nb, tok_pad, x_bf, egu, ed)


def _combine_kernel(pos_ref, twb_ref, y_ref, out_ref):
    tb = pl.program_id(0)

    def body(r, _):
        g = tb * TB + r
        p0 = pos_ref[2 * g]
        p1 = pos_ref[2 * g + 1]
        w0 = jax.lax.bitcast_convert_type(twb_ref[2 * g], jnp.float32)
        w1 = jax.lax.bitcast_convert_type(twb_ref[2 * g + 1], jnp.float32)
        out_ref[pl.ds(r, 1), :] = (y_ref[pl.ds(p0, 1), :] * w0
                                   + y_ref[pl.ds(p1, 1), :] * w1)
        return 0

    jax.lax.fori_loop(0, TB, body, 0)


def _combine_call(pos_flat, tw_flat_bits, y_pad, interpret=False):
    spec = pltpu.PrefetchScalarGridSpec(
        num_scalar_prefetch=2,
        grid=(T // TB,),
        in_specs=[pl.BlockSpec((NPAD, D), lambda tb, pos, tw: (0, 0))],
        out_specs=pl.BlockSpec((TB, D), lambda tb, pos, tw: (tb, 0)),
    )
    return pl.pallas_call(
        _combine_kernel,
        grid_spec=spec,
        out_shape=jax.ShapeDtypeStruct((T, D), jnp.float32),
        interpret=interpret,
    )(pos_flat, tw_flat_bits, y_pad)


def _moe(x, w_gate, egu, ed, interpret=False):
    b_, s_, d_ = x.shape
    x_flat = x.reshape(T, D)
    ti, tw, cnt, pos, aux = _router_call(x_flat, w_gate, interpret=interpret)

    pb = (cnt.reshape(E) + (BLK - 1)) // BLK
    ends = jnp.cumsum(pb)
    nblk_act = ends[E - 1].astype(jnp.int32).reshape(1)
    bids = jnp.arange(NBLK, dtype=jnp.int32)
    eb = jnp.minimum(jnp.sum((bids[:, None] >= ends[None, :]).astype(jnp.int32),
                             axis=1), E - 1).astype(jnp.int32)
    last_e = jnp.max(jnp.where(bids < nblk_act[0], eb, -1))
    be = jnp.where(bids < nblk_act[0], eb, last_e).astype(jnp.int32)
    pos_flat = pos.reshape(NP)
    tok_src = (jnp.arange(NP, dtype=jnp.int32) // K)
    tok_pad = jnp.zeros((NPAD,), jnp.int32).at[pos_flat].set(tok_src)

    y_pad = _ffn_call(be, nblk_act, tok_pad, x_flat, egu, ed,
                      interpret=interpret)

    tw_bits = jax.lax.bitcast_convert_type(tw.reshape(NP), jnp.int32)
    out = _combine_call(pos_flat, tw_bits, y_pad, interpret=interpret)
    return out.reshape(b_, s_, d_), aux[0, 0]


def kernel(x, W_gate, expert_gate_up, expert_down):
    return _moe(x, W_gate, expert_gate_up, expert_down)
